# baseline (speedup 1.0000x reference)
import jax
import jax.numpy as jnp
from jax.experimental import pallas as pl


def _gelu(x):
    return 0.5 * x * (1.0 + jax.lax.erf(x * (2.0 ** -0.5)))


def _xattn_kernel(adj_ref, sem_ref, w_ref, w1_ref, g_ref, b_ref, w2_ref,
                  out_ref):
    n = adj_ref.shape[0]
    d = w_ref.shape[0]
    sem0 = sem_ref[:, 0, :].reshape(n, d)
    h = jnp.dot(sem0, w_ref[:], preferred_element_type=jnp.float32)
    ones = jnp.ones((n, 1), jnp.float32)
    hx = jnp.concatenate([h, ones], axis=1)
    mask = (adj_ref[:] != 0).astype(jnp.float32)
    agg = jax.lax.dot_general(
        mask, hx, (((0,), (0,)), ((), ())),
        preferred_element_type=jnp.float32)
    sums = agg[:, :d]
    counts = agg[:, d:d + 1]
    x = _gelu(sums / jnp.maximum(counts, 1.0))
    x = jax.lax.dot_general(
        x, w1_ref[:], (((1,), (1,)), ((), ())),
        preferred_element_type=jnp.float32)
    x = _gelu(x)
    mu = jnp.mean(x, axis=-1, keepdims=True)
    var = jnp.mean((x - mu) ** 2, axis=-1, keepdims=True)
    x = (x - mu) / jnp.sqrt(var + 1e-5) * g_ref[:] + b_ref[:]
    out_ref[:] = jax.lax.dot_general(
        x, w2_ref[:], (((1,), (1,)), ((), ())),
        preferred_element_type=jnp.float32)


@jax.jit
def kernel(adj, semantics, attention_masks, W, W1, ln_g, ln_b, W2):
    del attention_masks
    n, seq, d = semantics.shape
    out = pl.pallas_call(
        _xattn_kernel,
        grid=(1,),
        in_specs=[
            pl.BlockSpec((n, n), lambda i: (0, 0)),
            pl.BlockSpec((n, 8, d), lambda i: (0, 0, 0)),
            pl.BlockSpec((d, d), lambda i: (0, 0)),
            pl.BlockSpec((d, d), lambda i: (0, 0)),
            pl.BlockSpec((1, d), lambda i: (0, 0)),
            pl.BlockSpec((1, d), lambda i: (0, 0)),
            pl.BlockSpec((1, d), lambda i: (0, 0)),
        ],
        out_specs=pl.BlockSpec((n, 1), lambda i: (0, 0)),
        out_shape=jax.ShapeDtypeStruct((n, 1), jnp.float32),
    )(adj, semantics, W, W1, ln_g.reshape(1, d), ln_b.reshape(1, d), W2)
    return out[:, 0]


# speedup vs baseline: 1.0042x; 1.0042x over previous
import jax
import jax.numpy as jnp
from jax.experimental import pallas as pl
from jax.experimental.pallas import tpu as pltpu

_NQ = 8


def _gelu(x):
    return 0.5 * x * (1.0 + jax.lax.erf(x * (2.0 ** -0.5)))


def _xattn_kernel(adj_hbm, sem_hbm, w_ref, w1_ref, g_ref, b_ref, w2_ref,
                  out_ref, adj_vmem, sem0_vmem, dma_sems):
    n = adj_vmem.shape[0]
    d = w_ref.shape[0]
    rows = n // _NQ
    adj_cp = pltpu.make_async_copy(adj_hbm, adj_vmem, dma_sems.at[_NQ])
    adj_cp.start()
    copies = []
    for q in range(_NQ):
        sl = pl.ds(q * rows, rows)
        cp = pltpu.make_async_copy(
            sem_hbm.at[sl, pl.ds(0, 1), :], sem0_vmem.at[sl, :, :],
            dma_sems.at[q])
        cp.start()
        copies.append(cp)
    for cp in copies:
        cp.wait()
    sem0 = sem0_vmem[:, 0, :].reshape(n, d)
    h = jnp.dot(sem0, w_ref[:], preferred_element_type=jnp.float32)
    ones = jnp.ones((n, 1), jnp.float32)
    hx = jnp.concatenate([h, ones], axis=1)
    adj_cp.wait()
    mask = (adj_vmem[:] != 0).astype(jnp.float32)
    agg = jax.lax.dot_general(
        mask, hx, (((0,), (0,)), ((), ())),
        preferred_element_type=jnp.float32)
    sums = agg[:, :d]
    counts = agg[:, d:d + 1]
    x = _gelu(sums / jnp.maximum(counts, 1.0))
    x = jax.lax.dot_general(
        x, w1_ref[:], (((1,), (1,)), ((), ())),
        preferred_element_type=jnp.float32)
    x = _gelu(x)
    mu = jnp.mean(x, axis=-1, keepdims=True)
    var = jnp.mean((x - mu) ** 2, axis=-1, keepdims=True)
    x = (x - mu) / jnp.sqrt(var + 1e-5) * g_ref[:] + b_ref[:]
    out_ref[:] = jax.lax.dot_general(
        x, w2_ref[:], (((1,), (1,)), ((), ())),
        preferred_element_type=jnp.float32)


@jax.jit
def kernel(adj, semantics, attention_masks, W, W1, ln_g, ln_b, W2):
    del attention_masks
    n, seq, d = semantics.shape
    out = pl.pallas_call(
        _xattn_kernel,
        grid=(1,),
        in_specs=[
            pl.BlockSpec(memory_space=pl.ANY),
            pl.BlockSpec(memory_space=pl.ANY),
            pl.BlockSpec((d, d), lambda i: (0, 0)),
            pl.BlockSpec((d, d), lambda i: (0, 0)),
            pl.BlockSpec((1, d), lambda i: (0, 0)),
            pl.BlockSpec((1, d), lambda i: (0, 0)),
            pl.BlockSpec((1, d), lambda i: (0, 0)),
        ],
        out_specs=pl.BlockSpec((n, 1), lambda i: (0, 0)),
        out_shape=jax.ShapeDtypeStruct((n, 1), jnp.float32),
        scratch_shapes=[
            pltpu.VMEM((n, n), jnp.float32),
            pltpu.VMEM((n, 1, d), jnp.float32),
            pltpu.SemaphoreType.DMA((_NQ + 1,)),
        ],
    )(adj, semantics, W, W1, ln_g.reshape(1, d), ln_b.reshape(1, d), W2)
    return out[:, 0]
